# baseline (speedup 1.0000x reference)
import dataclasses
import functools

import jax
import jax.numpy as jnp
import numpy as np
from jax import lax
from jax.experimental import pallas as pl
from jax.experimental.pallas import tpu as pltpu
from jax.experimental.pallas import tpu_sc as plsc

E = 16
L = 16
D = 2048
B = 4
H = 16
S = 4096
DH = D // H

_CHUNK = 256
_NCH = (L * D) // _CHUNK
_NW = 32
_ROWS_W = (B * _NCH) // _NW
_WPB = _NCH // _ROWS_W


def _sc_gather(ids16, ep_chunks):
    mesh = plsc.VectorSubcoreMesh(core_axis_name="c", subcore_axis_name="s")
    cp = pltpu.CompilerParams()
    if "needs_layout_passes" in pltpu.CompilerParams.__dataclass_fields__:
        cp = dataclasses.replace(cp, needs_layout_passes=False)

    @functools.partial(
        pl.kernel,
        out_type=jax.ShapeDtypeStruct((B * _NCH, _CHUNK), jnp.float32),
        mesh=mesh,
        compiler_params=cp,
        scratch_types=[
            pltpu.VMEM((16,), jnp.int32),
            pltpu.VMEM((16,), jnp.int32),
            pltpu.VMEM((_ROWS_W, _CHUNK), jnp.float32),
            pltpu.SemaphoreType.DMA,
        ],
    )
    def k(ids_hbm, ep_hbm, out_hbm, tid_v, idx_v, rows_v, sem):
        wid = lax.axis_index("s") * 2 + lax.axis_index("c")
        pltpu.sync_copy(ids_hbm, tid_v)
        b = wid // _WPB
        c0 = (wid % _WPB) * _ROWS_W
        bvec = jnp.full((16,), b, jnp.int32)
        tid_b = plsc.load_gather(tid_v, [bvec])
        idx_v[...] = tid_b * _NCH + c0 + lax.iota(jnp.int32, 16)
        pltpu.async_copy(ep_hbm.at[idx_v], rows_v, sem).wait()
        pltpu.sync_copy(rows_v, out_hbm.at[pl.ds(wid * _ROWS_W, _ROWS_W)])

    return k(ids16, ep_chunks)


_BN = 256


def _proj_body(x_ref, wk_ref, wv_ref, g_ref, k_ref, v_ref, sg_ref):
    x = x_ref[...]
    k_ref[...] = jnp.dot(x, wk_ref[...], preferred_element_type=jnp.float32)
    v_ref[...] = jnp.dot(x, wv_ref[...], preferred_element_type=jnp.float32)
    sg_ref[...] = jax.nn.sigmoid(g_ref[...])


def _proj(x2d, Wk, Wv, gates2d, interpret=False):
    return pl.pallas_call(
        _proj_body,
        grid=(D // _BN,),
        in_specs=[
            pl.BlockSpec((B * L, D), lambda j: (0, 0)),
            pl.BlockSpec((D, _BN), lambda j: (0, j)),
            pl.BlockSpec((D, _BN), lambda j: (0, j)),
            pl.BlockSpec((1, E), lambda j: (0, 0)),
        ],
        out_specs=[
            pl.BlockSpec((B * L, _BN), lambda j: (0, j)),
            pl.BlockSpec((B * L, _BN), lambda j: (0, j)),
            pl.BlockSpec((1, E), lambda j: (0, 0)),
        ],
        out_shape=[
            jax.ShapeDtypeStruct((B * L, D), jnp.float32),
            jax.ShapeDtypeStruct((B * L, D), jnp.float32),
            jax.ShapeDtypeStruct((1, E), jnp.float32),
        ],
        interpret=interpret,
    )(x2d, Wk, Wv, gates2d)


_SP = 16
_S16 = S // _SP
_PK = _SP * L
_PN = _SP * DH

_SEG = np.kron(np.eye(_SP, dtype=np.float32), np.ones((L, L), np.float32))
_VMASK = np.kron(np.eye(_SP, dtype=np.float32), np.ones((L, DH), np.float32))


def _attend_body(tid_ref, sg_ref, aw_ref, v_ref, seg_ref, mask_ref, o_ref):
    b = pl.program_id(0)
    g = sg_ref[0, tid_ref[b]]
    x = aw_ref[0, 0]
    e = jnp.exp(x)
    z = jnp.dot(e, seg_ref[...], preferred_element_type=jnp.float32)
    r = (e * (g / z)).astype(jnp.bfloat16)
    vb = v_ref[0, 0].astype(jnp.bfloat16)
    row = jnp.concatenate([vb] * _SP, axis=1)
    vbig = jnp.concatenate([row] * _SP, axis=0)
    vbig = vbig * mask_ref[...]
    o_ref[0, 0] = jnp.dot(r, vbig, preferred_element_type=jnp.float32)


def _attend(aw_packed, v_heads_t, sg, task_ids, seg, mask, interpret=False):
    return pl.pallas_call(
        _attend_body,
        grid=(B, H),
        in_specs=[
            pl.BlockSpec(memory_space=pltpu.SMEM),
            pl.BlockSpec(memory_space=pltpu.SMEM),
            pl.BlockSpec((1, 1, _S16, _PK), lambda b, h: (b, h, 0, 0)),
            pl.BlockSpec((1, 1, L, DH), lambda b, h: (b, h, 0, 0)),
            pl.BlockSpec((_PK, _PK), lambda b, h: (0, 0)),
            pl.BlockSpec((_PK, _PN), lambda b, h: (0, 0)),
        ],
        out_specs=pl.BlockSpec((1, 1, _S16, _PN), lambda b, h: (b, h, 0, 0)),
        out_shape=jax.ShapeDtypeStruct((B, H, _S16, _PN), jnp.float32),
        interpret=interpret,
    )(task_ids, sg, aw_packed, v_heads_t, seg, mask)


def kernel(task_ids, expert_prompts, Wk, Wv, gates, adapter_weights):
    task_ids = task_ids.astype(jnp.int32)
    ids16 = jnp.zeros((16,), jnp.int32).at[:B].set(task_ids)
    prompts_chunks = _sc_gather(ids16, expert_prompts.reshape(E * _NCH, _CHUNK))
    x2d = prompts_chunks.reshape(B * L, D)
    k2d, v2d, sg = _proj(x2d, Wk, Wv, gates.reshape(1, E))
    adapter_k = k2d.reshape(B, L, D)
    v_heads_t = jnp.transpose(v2d.reshape(B, L, H, DH), (0, 2, 1, 3))
    aw_packed = adapter_weights.reshape(B, H, _S16, _PK)
    seg = jnp.asarray(_SEG)
    mask = jnp.asarray(_VMASK, dtype=jnp.bfloat16)
    out_packed = _attend(aw_packed, v_heads_t, sg, task_ids, seg, mask)
    out = out_packed.reshape(B, H, S, DH)
    return out, adapter_k


# speedup vs baseline: 1.6173x; 1.6173x over previous
import dataclasses
import functools

import jax
import jax.numpy as jnp
from jax import lax
from jax.experimental import pallas as pl
from jax.experimental.pallas import tpu as pltpu
from jax.experimental.pallas import tpu_sc as plsc

E = 16
L = 16
D = 2048
B = 4
H = 16
S = 4096
DH = D // H


def _sc_gather(ids16, ep_rows):
    mesh = plsc.VectorSubcoreMesh(core_axis_name="c", subcore_axis_name="s")
    cp = pltpu.CompilerParams()
    if "needs_layout_passes" in pltpu.CompilerParams.__dataclass_fields__:
        cp = dataclasses.replace(cp, needs_layout_passes=False)

    @functools.partial(
        pl.kernel,
        out_type=jax.ShapeDtypeStruct((B * L, D), jnp.float32),
        mesh=mesh,
        compiler_params=cp,
        scratch_types=[
            pltpu.VMEM((16,), jnp.int32),
            pltpu.VMEM((16,), jnp.int32),
            pltpu.VMEM((L, D), jnp.float32),
            pltpu.SemaphoreType.DMA,
        ],
    )
    def k(ids_hbm, ep_hbm, out_hbm, tid_v, idx_v, rows_v, sem):
        wid = lax.axis_index("s") * 2 + lax.axis_index("c")

        @pl.when(wid < B)
        def _():
            pltpu.sync_copy(ids_hbm, tid_v)
            bvec = jnp.full((16,), wid, jnp.int32)
            tid_b = plsc.load_gather(tid_v, [bvec])
            idx_v[...] = tid_b * L + lax.iota(jnp.int32, 16)
            pltpu.async_copy(ep_hbm.at[idx_v], rows_v, sem).wait()
            pltpu.sync_copy(rows_v, out_hbm.at[pl.ds(wid * L, L)])

    return k(ids16, ep_rows)


_BN = 128


def _proj_body(x_ref, wk_ref, wv_ref, g_ref, k_ref, v_ref, sg_ref):
    x = x_ref[...]
    k_ref[...] = jnp.dot(x, wk_ref[...], preferred_element_type=jnp.float32)
    v = jnp.dot(x, wv_ref[...], preferred_element_type=jnp.float32)
    v_ref[...] = v.reshape(B, 1, L, DH)
    sg_ref[...] = jax.nn.sigmoid(g_ref[...])


def _proj(x2d, Wk, Wv, gates2d, interpret=False):
    return pl.pallas_call(
        _proj_body,
        grid=(D // _BN,),
        in_specs=[
            pl.BlockSpec((B * L, D), lambda j: (0, 0)),
            pl.BlockSpec((D, _BN), lambda j: (0, j)),
            pl.BlockSpec((D, _BN), lambda j: (0, j)),
            pl.BlockSpec((1, E), lambda j: (0, 0)),
        ],
        out_specs=[
            pl.BlockSpec((B * L, _BN), lambda j: (0, j)),
            pl.BlockSpec((B, 1, L, DH), lambda j: (0, j, 0, 0)),
            pl.BlockSpec((1, E), lambda j: (0, 0)),
        ],
        out_shape=[
            jax.ShapeDtypeStruct((B * L, D), jnp.float32),
            jax.ShapeDtypeStruct((B, H, L, DH), jnp.float32),
            jax.ShapeDtypeStruct((1, E), jnp.float32),
        ],
        interpret=interpret,
    )(x2d, Wk, Wv, gates2d)


def _attend_body(tid_ref, sg_ref, aw_ref, v_ref, o_ref):
    b = pl.program_id(0)
    g = sg_ref[0, tid_ref[b]]
    x = aw_ref[0, 0]
    xt = x.T
    e = jnp.exp(xt)
    z = jnp.sum(e, axis=0, keepdims=True)
    r = (e * (g / z)).astype(jnp.bfloat16)
    vb = v_ref[0, 0].astype(jnp.bfloat16)
    o_ref[0, 0] = lax.dot_general(
        r, vb, (((0,), (0,)), ((), ())),
        preferred_element_type=jnp.float32,
    )


def _attend(aw, v_heads_t, sg, task_ids, interpret=False):
    return pl.pallas_call(
        _attend_body,
        grid=(B, H),
        in_specs=[
            pl.BlockSpec(memory_space=pltpu.SMEM),
            pl.BlockSpec(memory_space=pltpu.SMEM),
            pl.BlockSpec((1, 1, S, L), lambda b, h: (b, h, 0, 0)),
            pl.BlockSpec((1, 1, L, DH), lambda b, h: (b, h, 0, 0)),
        ],
        out_specs=pl.BlockSpec((1, 1, S, DH), lambda b, h: (b, h, 0, 0)),
        out_shape=jax.ShapeDtypeStruct((B, H, S, DH), jnp.float32),
        interpret=interpret,
    )(task_ids, sg, aw, v_heads_t)


def kernel(task_ids, expert_prompts, Wk, Wv, gates, adapter_weights):
    task_ids = task_ids.astype(jnp.int32)
    ids16 = jnp.zeros((16,), jnp.int32).at[:B].set(task_ids)
    x2d = _sc_gather(ids16, expert_prompts.reshape(E * L, D))
    k2d, v_heads_t, sg = _proj(x2d, Wk, Wv, gates.reshape(1, E))
    adapter_k = k2d.reshape(B, L, D)
    out = _attend(adapter_weights, v_heads_t, sg, task_ids)
    return out, adapter_k


# speedup vs baseline: 1.6503x; 1.0204x over previous
import dataclasses
import functools

import jax
import jax.numpy as jnp
from jax import lax
from jax.experimental import pallas as pl
from jax.experimental.pallas import tpu as pltpu
from jax.experimental.pallas import tpu_sc as plsc

E = 16
L = 16
D = 2048
B = 4
H = 16
S = 4096
DH = D // H


def _sc_gather(ids16, ep_rows):
    mesh = plsc.VectorSubcoreMesh(core_axis_name="c", subcore_axis_name="s")
    cp = pltpu.CompilerParams()
    if "needs_layout_passes" in pltpu.CompilerParams.__dataclass_fields__:
        cp = dataclasses.replace(cp, needs_layout_passes=False)

    @functools.partial(
        pl.kernel,
        out_type=jax.ShapeDtypeStruct((B * L, D), jnp.float32),
        mesh=mesh,
        compiler_params=cp,
        scratch_types=[
            pltpu.VMEM((16,), jnp.int32),
            pltpu.VMEM((16,), jnp.int32),
            pltpu.VMEM((L, D), jnp.float32),
            pltpu.SemaphoreType.DMA,
        ],
    )
    def k(ids_hbm, ep_hbm, out_hbm, tid_v, idx_v, rows_v, sem):
        wid = lax.axis_index("s") * 2 + lax.axis_index("c")

        @pl.when(wid < B)
        def _():
            pltpu.sync_copy(ids_hbm, tid_v)
            bvec = jnp.full((16,), wid, jnp.int32)
            tid_b = plsc.load_gather(tid_v, [bvec])
            idx_v[...] = tid_b * L + lax.iota(jnp.int32, 16)
            pltpu.async_copy(ep_hbm.at[idx_v], rows_v, sem).wait()
            pltpu.sync_copy(rows_v, out_hbm.at[pl.ds(wid * L, L)])

    return k(ids16, ep_rows)


_BK = 256


def _proj_body(x_ref, wk_ref, wv_ref, g_ref, k_ref, v_ref, sg_ref):
    j = pl.program_id(0)
    x = x_ref[...]
    pk = jnp.dot(x, wk_ref[...], preferred_element_type=jnp.float32)
    pv = jnp.dot(x, wv_ref[...], preferred_element_type=jnp.float32)

    @pl.when(j == 0)
    def _():
        k_ref[...] = pk
        v_ref[...] = pv
        sg_ref[...] = jax.nn.sigmoid(g_ref[...])

    @pl.when(j > 0)
    def _():
        k_ref[...] += pk
        v_ref[...] += pv


def _proj(x2d, Wk, Wv, gates2d, interpret=False):
    return pl.pallas_call(
        _proj_body,
        grid=(D // _BK,),
        in_specs=[
            pl.BlockSpec((B * L, _BK), lambda j: (0, j)),
            pl.BlockSpec((_BK, D), lambda j: (j, 0)),
            pl.BlockSpec((_BK, D), lambda j: (j, 0)),
            pl.BlockSpec((1, E), lambda j: (0, 0)),
        ],
        out_specs=[
            pl.BlockSpec((B * L, D), lambda j: (0, 0)),
            pl.BlockSpec((B * L, D), lambda j: (0, 0)),
            pl.BlockSpec((1, E), lambda j: (0, 0)),
        ],
        out_shape=[
            jax.ShapeDtypeStruct((B * L, D), jnp.float32),
            jax.ShapeDtypeStruct((B * L, D), jnp.float32),
            jax.ShapeDtypeStruct((1, E), jnp.float32),
        ],
        interpret=interpret,
    )(x2d, Wk, Wv, gates2d)


def _attend_body(tid_ref, sg_ref, aw_ref, v_ref, o_ref):
    b = pl.program_id(0)
    g = sg_ref[0, tid_ref[b]]
    x = aw_ref[0, 0]
    xt = x.T
    e = jnp.exp(xt)
    z = jnp.sum(e, axis=0, keepdims=True)
    r = (e * (g / z)).astype(jnp.bfloat16)
    vb = v_ref[...].astype(jnp.bfloat16)
    o_ref[0, 0] = lax.dot_general(
        r, vb, (((0,), (0,)), ((), ())),
        preferred_element_type=jnp.float32,
    )


def _attend(aw, v2d, sg, task_ids, interpret=False):
    return pl.pallas_call(
        _attend_body,
        grid=(B, H),
        in_specs=[
            pl.BlockSpec(memory_space=pltpu.SMEM),
            pl.BlockSpec(memory_space=pltpu.SMEM),
            pl.BlockSpec((1, 1, S, L), lambda b, h: (b, h, 0, 0)),
            pl.BlockSpec((L, DH), lambda b, h: (b, h)),
        ],
        out_specs=pl.BlockSpec((1, 1, S, DH), lambda b, h: (b, h, 0, 0)),
        out_shape=jax.ShapeDtypeStruct((B, H, S, DH), jnp.float32),
        interpret=interpret,
    )(task_ids, sg, aw, v2d)


def kernel(task_ids, expert_prompts, Wk, Wv, gates, adapter_weights):
    task_ids = task_ids.astype(jnp.int32)
    ids16 = jnp.zeros((16,), jnp.int32).at[:B].set(task_ids)
    x2d = _sc_gather(ids16, expert_prompts.reshape(E * L, D))
    k2d, v2d, sg = _proj(x2d, Wk, Wv, gates.reshape(1, E))
    adapter_k = k2d.reshape(B, L, D)
    out = _attend(adapter_weights, v2d, sg, task_ids)
    return out, adapter_k


# speedup vs baseline: 3.2209x; 1.9517x over previous
import dataclasses
import functools

import jax
import jax.numpy as jnp
from jax import lax
from jax.experimental import pallas as pl
from jax.experimental.pallas import tpu as pltpu
from jax.experimental.pallas import tpu_sc as plsc

E = 16
L = 16
D = 2048
B = 4
H = 16
S = 4096
DH = D // H


def _sc_gather(ids16, ep_rows):
    mesh = plsc.VectorSubcoreMesh(core_axis_name="c", subcore_axis_name="s")
    cp = pltpu.CompilerParams()
    if "needs_layout_passes" in pltpu.CompilerParams.__dataclass_fields__:
        cp = dataclasses.replace(cp, needs_layout_passes=False)

    @functools.partial(
        pl.kernel,
        out_type=jax.ShapeDtypeStruct((B * L, D), jnp.float32),
        mesh=mesh,
        compiler_params=cp,
        scratch_types=[
            pltpu.VMEM((16,), jnp.int32),
            pltpu.VMEM((16,), jnp.int32),
            pltpu.VMEM((L, D), jnp.float32),
            pltpu.SemaphoreType.DMA,
        ],
    )
    def k(ids_hbm, ep_hbm, out_hbm, tid_v, idx_v, rows_v, sem):
        wid = lax.axis_index("s") * 2 + lax.axis_index("c")

        @pl.when(wid < B)
        def _():
            pltpu.sync_copy(ids_hbm, tid_v)
            bvec = jnp.full((16,), wid, jnp.int32)
            tid_b = plsc.load_gather(tid_v, [bvec])
            idx_v[...] = tid_b * L + lax.iota(jnp.int32, 16)
            pltpu.async_copy(ep_hbm.at[idx_v], rows_v, sem).wait()
            pltpu.sync_copy(rows_v, out_hbm.at[pl.ds(wid * L, L)])

    return k(ids16, ep_rows)


_BK = 256


def _proj_body(x_ref, wk_ref, wv_ref, g_ref, k_ref, v_ref, sg_ref):
    j = pl.program_id(0)
    x = x_ref[...]
    pk = jnp.dot(x, wk_ref[...], preferred_element_type=jnp.float32)
    pv = jnp.dot(x, wv_ref[...], preferred_element_type=jnp.float32)

    @pl.when(j == 0)
    def _():
        k_ref[...] = pk
        v_ref[...] = pv
        sg_ref[...] = jax.nn.sigmoid(g_ref[...])

    @pl.when(j > 0)
    def _():
        k_ref[...] += pk
        v_ref[...] += pv


def _proj(x2d, Wk, Wv, gates2d, interpret=False):
    return pl.pallas_call(
        _proj_body,
        grid=(D // _BK,),
        in_specs=[
            pl.BlockSpec((B * L, _BK), lambda j: (0, j)),
            pl.BlockSpec((_BK, D), lambda j: (j, 0)),
            pl.BlockSpec((_BK, D), lambda j: (j, 0)),
            pl.BlockSpec((1, E), lambda j: (0, 0)),
        ],
        out_specs=[
            pl.BlockSpec((B * L, D), lambda j: (0, 0)),
            pl.BlockSpec((B * L, D), lambda j: (0, 0)),
            pl.BlockSpec((1, E), lambda j: (0, 0)),
        ],
        out_shape=[
            jax.ShapeDtypeStruct((B * L, D), jnp.float32),
            jax.ShapeDtypeStruct((B * L, D), jnp.float32),
            jax.ShapeDtypeStruct((1, E), jnp.float32),
        ],
        interpret=interpret,
    )(x2d, Wk, Wv, gates2d)


def _attend_body(tid_ref, sg_ref, aw_ref, v_ref, o_ref):
    b = pl.program_id(0)
    g = sg_ref[0, tid_ref[b]]
    xt = aw_ref[0, 0]
    e = jnp.exp(xt)
    z = jnp.sum(e, axis=0, keepdims=True)
    r = (e * (g / z)).astype(jnp.bfloat16)
    vb = v_ref[...].astype(jnp.bfloat16)
    o_ref[0, 0] = lax.dot_general(
        r, vb, (((0,), (0,)), ((), ())),
        preferred_element_type=jnp.float32,
    )


def _attend(aw_t, v2d, sg, task_ids, interpret=False):
    return pl.pallas_call(
        _attend_body,
        grid=(B, H),
        in_specs=[
            pl.BlockSpec(memory_space=pltpu.SMEM),
            pl.BlockSpec(memory_space=pltpu.SMEM),
            pl.BlockSpec((1, 1, L, S), lambda b, h: (b, h, 0, 0)),
            pl.BlockSpec((L, DH), lambda b, h: (b, h)),
        ],
        out_specs=pl.BlockSpec((1, 1, S, DH), lambda b, h: (b, h, 0, 0)),
        out_shape=jax.ShapeDtypeStruct((B, H, S, DH), jnp.float32),
        compiler_params=pltpu.CompilerParams(
            dimension_semantics=("parallel", "parallel"),
        ),
        interpret=interpret,
    )(task_ids, sg, aw_t, v2d)


def kernel(task_ids, expert_prompts, Wk, Wv, gates, adapter_weights):
    task_ids = task_ids.astype(jnp.int32)
    ids16 = jnp.zeros((16,), jnp.int32).at[:B].set(task_ids)
    x2d = _sc_gather(ids16, expert_prompts.reshape(E * L, D))
    k2d, v2d, sg = _proj(x2d, Wk, Wv, gates.reshape(1, E))
    adapter_k = k2d.reshape(B, L, D)
    aw_t = jnp.swapaxes(adapter_weights, 2, 3)
    out = _attend(aw_t, v2d, sg, task_ids)
    return out, adapter_k
